# baseline (speedup 1.0000x reference)
import functools

import jax
import jax.numpy as jnp
from jax import lax
from jax.experimental import pallas as pl
from jax.experimental.pallas import tpu as pltpu
from jax.experimental.pallas import tpu_sc as plsc

F32 = jnp.float32
I32 = jnp.int32
H = 128
_NG = 2500
NW = 32
CH = 80
K = 2
_GRAN = NW * CH * 2 * K


def _silu(x):
    return x * jax.nn.sigmoid(x)


def _ln(h, g, b):
    mu = jnp.mean(h, axis=-1, keepdims=True)
    var = jnp.mean((h - mu) ** 2, axis=-1, keepdims=True)
    return (h - mu) * lax.rsqrt(var + 1e-5) * g + b


def _pick_rows(n, cap=2048):
    best = None
    for cand in range(8, min(n, cap) + 1, 8):
        if n % cand == 0:
            best = cand
    return best if best is not None else n


def _mm(a, b):
    return jnp.dot(a, b, preferred_element_type=F32)


def _rowwise(fn, xs, ws, out_widths, R=None):
    E = xs[0].shape[0]
    if R is None:
        R = _pick_rows(E)
    grid = (E // R,)
    nx, nw = len(xs), len(ws)
    in_specs = [pl.BlockSpec((R, x.shape[1]), lambda i: (i, 0)) for x in xs]
    in_specs += [pl.BlockSpec(w.shape, lambda i: (0, 0)) for w in ws]
    out_specs = [pl.BlockSpec((R, wd), lambda i: (i, 0)) for wd in out_widths]
    out_shape = [jax.ShapeDtypeStruct((E, wd), F32) for wd in out_widths]

    def body(*refs):
        xr = [r[...] for r in refs[:nx]]
        wr = [r[...] for r in refs[nx:nx + nw]]
        outs = fn(*xr, *wr)
        if not isinstance(outs, tuple):
            outs = (outs,)
        for o_ref, o in zip(refs[nx + nw:], outs):
            o_ref[...] = o

    return pl.pallas_call(body, grid=grid, in_specs=in_specs,
                          out_specs=out_specs, out_shape=out_shape)(*xs, *ws)


def _ffb_weights(p):
    ws = [p["W1"], p["b1"].reshape(1, -1), p["W2"], p["b2"].reshape(1, -1)]
    if "g" in p:
        ws += [p["g"].reshape(1, -1), p["beta"].reshape(1, -1)]
    return ws


def _ffb(x, p, act=_silu):
    has_ln = "g" in p

    def fn(xv, w1, b1, w2, b2, *rest):
        h = act(_mm(xv, w1) + b1)
        h = _mm(h, w2) + b2
        if has_ln:
            h = _ln(h, rest[0], rest[1])
        return h

    return _rowwise(fn, [x], _ffb_weights(p), [p["W2"].shape[1]])[0]


def _edge_weights(p):
    w1 = p["W1"]
    return [w1[:H], w1[H:2 * H], w1[2 * H:], p["b1"].reshape(1, -1),
            p["W2"], p["b2"].reshape(1, -1),
            p["g"].reshape(1, -1), p["beta"].reshape(1, -1)]


def _edge_mlp(ns, nr, ef, p):
    def fn(a, b, c, w1a, w1b, w1c, b1, w2, b2, g, beta):
        h = _silu(_mm(a, w1a) + _mm(b, w1b) + _mm(c, w1c) + b1)
        return _ln(_mm(h, w2) + b2, g, beta)

    return _rowwise(fn, [ns, nr, ef], _edge_weights(p), [H])[0]


def _proc_edge_mlp(ns, nr, pe, p):
    def fn(a, b, c, w1a, w1b, w1c, b1, w2, b2, g, beta):
        h = _silu(_mm(a, w1a) + _mm(b, w1b) + _mm(c, w1c) + b1)
        ne = _ln(_mm(h, w2) + b2, g, beta)
        return ne, c + ne

    return _rowwise(fn, [ns, nr, pe], _edge_weights(p), [H, H])


def _node2_weights(p):
    w1 = p["W1"]
    return [w1[:H], w1[H:], p["b1"].reshape(1, -1), p["W2"],
            p["b2"].reshape(1, -1), p["g"].reshape(1, -1),
            p["beta"].reshape(1, -1)]


def _enc_node(n0, p0, p1, cnt, pn, po):
    def fn(nv, a0, a1, cv, w1a, w1b, b1, w2, b2, g, beta,
           w1o, b1o, w2o, b2o):
        agg = (a0 + a1) / jnp.maximum(cv, 1.0)
        h = _silu(_mm(nv, w1a) + _mm(agg, w1b) + b1)
        v = nv + _ln(_mm(h, w2) + b2, g, beta)
        h2 = _silu(_mm(v, w1o) + b1o)
        return _mm(h2, w2o) + b2o

    ws = _node2_weights(pn) + [po["W1"], po["b1"].reshape(1, -1),
                               po["W2"], po["b2"].reshape(1, -1)]
    return _rowwise(fn, [n0, p0, p1, cnt], ws, [H])[0]


def _proc_node(nl, p0, p1, cnt, pn):
    def fn(nv, a0, a1, cv, w1a, w1b, b1, w2, b2, g, beta):
        agg = (a0 + a1) / jnp.maximum(cv, 1.0)
        h = _silu(_mm(nv, w1a) + _mm(agg, w1b) + b1)
        return nv + _ln(_mm(h, w2) + b2, g, beta)

    return _rowwise(fn, [nl, p0, p1, cnt], _node2_weights(pn), [H])[0]


def _dec_node(n, p0, p1, cnt, pn, po):
    dout = po["W2"].shape[1]

    def fn(nv, a0, a1, cv, w1a, w1b, b1, w2, b2, g, beta,
           w1o, b1o, w2o, b2o):
        agg = (a0 + a1) / jnp.maximum(cv, 1.0)
        h = _silu(_mm(nv, w1a) + _mm(agg, w1b) + b1)
        nn2 = _ln(_mm(h, w2) + b2, g, beta)
        h2 = jax.nn.sigmoid(_mm(nn2, w1o) + b1o)
        return _mm(h2, w2o) + b2o

    ws = _node2_weights(pn) + [po["W1"], po["b1"].reshape(1, -1),
                               po["W2"], po["b2"].reshape(1, -1)]
    return _rowwise(fn, [n, p0, p1, cnt], ws, [dout])[0]


def _sc_mesh():
    return plsc.VectorSubcoreMesh(core_axis_name="c", subcore_axis_name="s")


def _sc_gather_pair(t1, i1_3d, t2, i2_3d):
    nch = i1_3d.shape[1]
    rows = nch * CH
    E = NW * rows
    assert nch % (2 * K) == 0, nch
    i1f = i1_3d.reshape(-1)
    i2f = i2_3d.reshape(-1)

    scratch = [pltpu.VMEM((CH,), I32), pltpu.VMEM((CH,), I32),
               pltpu.VMEM((CH, H), F32), pltpu.VMEM((CH, H), F32)]
    scratch += [pltpu.SemaphoreType.DMA] * 2

    @functools.partial(
        pl.kernel, mesh=_sc_mesh(),
        out_type=(jax.ShapeDtypeStruct((E, H), F32),
                  jax.ShapeDtypeStruct((E, H), F32)),
        scratch_types=scratch,
    )
    def k(t1r, i1r, t2r, i2r, o1r, o2r, iv1, iv2, rb1, rb2, sm1, sm2):
        wid = lax.axis_index("s") * 2 + lax.axis_index("c")

        def body(ci, carry):
            base = wid * rows + ci * CH
            pltpu.sync_copy(i1r.at[pl.ds(base, CH)], iv1)
            pltpu.sync_copy(i2r.at[pl.ds(base, CH)], iv2)
            c1 = pltpu.async_copy(t1r.at[iv1], rb1, sm1)
            c2 = pltpu.async_copy(t2r.at[iv2], rb2, sm2)
            c1.wait()
            c2.wait()
            pltpu.sync_copy(rb1, o1r.at[pl.ds(base, CH)])
            pltpu.sync_copy(rb2, o2r.at[pl.ds(base, CH)])
            return carry

        lax.fori_loop(0, nch, body, 0)

    return k(t1, i1f, t2, i2f)


def _sc_scatter(vals, idx_flat, npad):
    E = idx_flat.shape[0]
    rows = E // NW
    nch = rows // CH
    assert vals.shape[0] == E and npad % 128 == 0, (vals.shape, E, npad)
    assert nch % 2 == 0, nch
    G2 = nch // 2
    zr = npad // 16
    zrows = jnp.zeros((zr, H), F32)
    ones = jnp.ones((CH, H), F32)

    scratch = [pltpu.VMEM((CH, H), F32) for _ in range(2)]
    scratch += [pltpu.VMEM((CH,), I32) for _ in range(2)]
    scratch += [pltpu.VMEM((CH, H), F32)]
    scratch += [pltpu.VMEM_SHARED((npad, H), F32)]
    scratch += [pltpu.SemaphoreType.DMA] * 4

    @functools.partial(
        pl.kernel, mesh=_sc_mesh(),
        out_type=(jax.ShapeDtypeStruct((2, npad, H), F32),
                  jax.ShapeDtypeStruct((2, npad, H), F32)),
        scratch_types=scratch,
    )
    def k(valsr, idxr, zrr, onesr, sumsr, cntr, *rest):
        vb = rest[0:2]
        ivc = rest[2:4]
        ob = rest[4]
        acc = rest[5]
        smv = rest[6:8]
        sma = rest[8:10]
        cid = lax.axis_index("c")
        sid = lax.axis_index("s")
        wid = sid * 2 + cid
        base0 = wid * rows

        def zero_acc():
            pltpu.sync_copy(zrr, acc.at[pl.ds(sid * zr, zr)])

        def fire_v(p, c, with_vals):
            off = base0 + c * CH
            if with_vals:
                pltpu.async_copy(valsr.at[pl.ds(off, CH)], vb[p], smv[p])
            pltpu.async_copy(idxr.at[pl.ds(off, CH)], ivc[p], smv[p])

        def wait_v(p, with_vals):
            if with_vals:
                pltpu.make_async_copy(valsr.at[pl.ds(base0, CH)], vb[p],
                                      smv[p]).wait()
            pltpu.make_async_copy(idxr.at[pl.ds(base0, CH)], ivc[p],
                                  smv[p]).wait()

        def fire_a(p, src):
            pltpu.async_copy(src, acc.at[ivc[p]], sma[p], add=True)

        def wait_a(p, src):
            pltpu.make_async_copy(src, acc.at[ivc[p]], sma[p]).wait()

        def phase(with_vals, outr):
            zero_acc()
            plsc.subcore_barrier()
            src = (lambda p: vb[p]) if with_vals else (lambda p: ob)
            fire_v(0, 0, with_vals)

            def body(i, carry):
                a = 2 * i
                wait_v(0, with_vals)
                fire_v(1, a + 1, with_vals)
                fire_a(0, src(0))
                wait_v(1, with_vals)
                wait_a(0, src(0))

                @pl.when(i + 1 < G2)
                def _():
                    fire_v(0, a + 2, with_vals)

                fire_a(1, src(1))
                wait_a(1, src(1))
                return carry

            lax.fori_loop(0, G2, body, 0)
            plsc.subcore_barrier()
            pltpu.sync_copy(acc.at[pl.ds(sid * zr, zr)],
                            outr.at[cid, pl.ds(sid * zr, zr)])

        pltpu.sync_copy(onesr, ob)
        phase(True, sumsr)
        plsc.subcore_barrier()
        phase(False, cntr)

    return k(vals, idx_flat, zrows, ones)


def _unpack_counts(cntp, n):
    return cntp[0, :n, :1] + cntp[1, :n, :1]


def _pad_to(x, n, value):
    if x.shape[0] == n:
        return x
    pad = [(0, n - x.shape[0])] + [(0, 0)] * (x.ndim - 1)
    return jnp.pad(x, pad, constant_values=value)


def _idx3(col, ep, fill):
    return _pad_to(col, ep, fill).reshape(NW, ep // (NW * CH), CH)


def kernel(pc2g_edge_idx, pc2g_edge_features, pc2g_node_features,
           g2g_edge_idx, g2g_edge_features, g2pc_edge_idx, g2pc_edge_features,
           params):
    NPC = pc2g_node_features.shape[0]
    NG = _NG
    NGP = ((NG + 1 + 127) // 128) * 128
    NPCP = ((NPC + 1 + 127) // 128) * 128
    E1 = pc2g_edge_idx.shape[0]
    E2 = g2g_edge_idx.shape[0]
    E3 = g2pc_edge_idx.shape[0]
    E1P = ((E1 + _GRAN - 1) // _GRAN) * _GRAN
    E2P = ((E2 + _GRAN - 1) // _GRAN) * _GRAN
    E3P = ((E3 + _GRAN - 1) // _GRAN) * _GRAN
    gmod = globals()
    gather_pair = gmod["_sc_gather_pair"]
    scatter = gmod["_sc_scatter"]

    enc, proc, dec = params["enc"], params["proc"], params["dec"]

    e1 = _ffb(_pad_to(pc2g_edge_features, E1P, 0.0), enc["embed_edge"])
    n_full = _ffb(_pad_to(pc2g_node_features, NPCP, 0.0), enc["embed_node"])
    s1g = _idx3(pc2g_edge_idx[:, 0], E1P, 0)
    r1g = _idx3(pc2g_edge_idx[:, 1], E1P, NG)
    r1f = _pad_to(pc2g_edge_idx[:, 1], E1P, NG)
    ns, nr = gather_pair(n_full, s1g, n_full, r1g)
    e2 = _edge_mlp(ns, nr, e1, enc["gn_edge"])
    sums, cntp1 = scatter(e2, r1f, NGP)
    n_lat = _enc_node(n_full[:NGP], sums[0], sums[1],
                      _unpack_counts(cntp1, NGP),
                      enc["gn_node"], enc["out"])

    pe = _ffb(_pad_to(g2g_edge_features, E2P, 0.0), proc["embed_edge"])
    s2g = _idx3(g2g_edge_idx[:, 0], E2P, NG)
    r2g = _idx3(g2g_edge_idx[:, 1], E2P, NG)
    r2f = _pad_to(g2g_edge_idx[:, 1], E2P, NG)

    def round_fn(carry, gp):
        nl, pev = carry
        ns2, nr2 = gather_pair(nl, s2g, nl, r2g)
        ne, pev = _proc_edge_mlp(ns2, nr2, pev, gp["edge"])
        sums2, cntp2 = scatter(ne, r2f, NGP)
        nl = _proc_node(nl, sums2[0], sums2[1],
                        _unpack_counts(cntp2, NGP), gp["node"])
        return (nl, pev), None

    stacked = jax.tree.map(lambda *xs: jnp.stack(xs), *proc["gn"])
    (n_lat, pe), _ = lax.scan(round_fn, (n_lat, pe), stacked)
    n_lat = _ffb(n_lat, proc["out"])

    de = _ffb(_pad_to(g2pc_edge_features, E3P, 0.0), dec["embed_edge"])
    s3g = _idx3(g2pc_edge_idx[:, 0], E3P, NG)
    r3g = _idx3(g2pc_edge_idx[:, 1], E3P, NPC)
    r3f = _pad_to(g2pc_edge_idx[:, 1], E3P, NPC)
    ns3, nr3 = gather_pair(n_lat, s3g, n_full, r3g)
    de2 = _edge_mlp(ns3, nr3, de, dec["proc_edge"])
    sums3, cntp3 = scatter(de2, r3f, NPCP)
    out = _dec_node(n_full[:NPC], sums3[0, :NPC], sums3[1, :NPC],
                    _unpack_counts(cntp3, NPC),
                    dec["proc_node"], dec["out"])
    return out


# speedup vs baseline: 1.6013x; 1.6013x over previous
import functools

import jax
import jax.numpy as jnp
from jax import lax
from jax.experimental import pallas as pl
from jax.experimental.pallas import tpu as pltpu
from jax.experimental.pallas import tpu_sc as plsc

F32 = jnp.float32
I32 = jnp.int32
H = 128
_NG = 2500
NW = 32
CH = 80
K = 2
_GRAN = NW * CH


def _silu(x):
    return x * jax.nn.sigmoid(x)


def _ln(h, g, b):
    mu = jnp.mean(h, axis=-1, keepdims=True)
    var = jnp.mean((h - mu) ** 2, axis=-1, keepdims=True)
    return (h - mu) * lax.rsqrt(var + 1e-5) * g + b


def _pick_rows(n, cap=2048):
    best = None
    for cand in range(8, min(n, cap) + 1, 8):
        if n % cand == 0:
            best = cand
    return best if best is not None else n


def _mm(a, b):
    return jnp.dot(a, b, preferred_element_type=F32)


def _rowwise(fn, xs, ws, out_widths, R=None):
    E = xs[0].shape[0]
    if R is None:
        R = _pick_rows(E)
    grid = (E // R,)
    nx, nw = len(xs), len(ws)
    in_specs = [pl.BlockSpec((R, x.shape[1]), lambda i: (i, 0)) for x in xs]
    in_specs += [pl.BlockSpec(w.shape, lambda i: (0, 0)) for w in ws]
    out_specs = [pl.BlockSpec((R, wd), lambda i: (i, 0)) for wd in out_widths]
    out_shape = [jax.ShapeDtypeStruct((E, wd), F32) for wd in out_widths]

    def body(*refs):
        xr = [r[...] for r in refs[:nx]]
        wr = [r[...] for r in refs[nx:nx + nw]]
        outs = fn(*xr, *wr)
        if not isinstance(outs, tuple):
            outs = (outs,)
        for o_ref, o in zip(refs[nx + nw:], outs):
            o_ref[...] = o

    return pl.pallas_call(body, grid=grid, in_specs=in_specs,
                          out_specs=out_specs, out_shape=out_shape)(*xs, *ws)


def _ffb_weights(p):
    ws = [p["W1"], p["b1"].reshape(1, -1), p["W2"], p["b2"].reshape(1, -1)]
    if "g" in p:
        ws += [p["g"].reshape(1, -1), p["beta"].reshape(1, -1)]
    return ws


def _ffb(x, p, act=_silu):
    has_ln = "g" in p

    def fn(xv, w1, b1, w2, b2, *rest):
        h = act(_mm(xv, w1) + b1)
        h = _mm(h, w2) + b2
        if has_ln:
            h = _ln(h, rest[0], rest[1])
        return h

    return _rowwise(fn, [x], _ffb_weights(p), [p["W2"].shape[1]])[0]


def _edge_weights(p):
    w1 = p["W1"]
    return [w1[:H], w1[H:2 * H], w1[2 * H:], p["b1"].reshape(1, -1),
            p["W2"], p["b2"].reshape(1, -1),
            p["g"].reshape(1, -1), p["beta"].reshape(1, -1)]


def _edge_mlp(ns, nr, ef, p):
    def fn(a, b, c, w1a, w1b, w1c, b1, w2, b2, g, beta):
        h = _silu(_mm(a, w1a) + _mm(b, w1b) + _mm(c, w1c) + b1)
        return _ln(_mm(h, w2) + b2, g, beta)

    return _rowwise(fn, [ns, nr, ef], _edge_weights(p), [H])[0]


def _proc_edge_mlp(ns, nr, pe, p):
    def fn(a, b, c, w1a, w1b, w1c, b1, w2, b2, g, beta):
        h = _silu(_mm(a, w1a) + _mm(b, w1b) + _mm(c, w1c) + b1)
        ne = _ln(_mm(h, w2) + b2, g, beta)
        return ne, c + ne

    return _rowwise(fn, [ns, nr, pe], _edge_weights(p), [H, H])


def _node2_weights(p):
    w1 = p["W1"]
    return [w1[:H], w1[H:], p["b1"].reshape(1, -1), p["W2"],
            p["b2"].reshape(1, -1), p["g"].reshape(1, -1),
            p["beta"].reshape(1, -1)]


def _enc_node(n0, p0, p1, cnt, pn, po):
    def fn(nv, a0, a1, cv, w1a, w1b, b1, w2, b2, g, beta,
           w1o, b1o, w2o, b2o):
        agg = (a0 + a1) / jnp.maximum(cv, 1.0)
        h = _silu(_mm(nv, w1a) + _mm(agg, w1b) + b1)
        v = nv + _ln(_mm(h, w2) + b2, g, beta)
        h2 = _silu(_mm(v, w1o) + b1o)
        return _mm(h2, w2o) + b2o

    ws = _node2_weights(pn) + [po["W1"], po["b1"].reshape(1, -1),
                               po["W2"], po["b2"].reshape(1, -1)]
    return _rowwise(fn, [n0, p0, p1, cnt], ws, [H])[0]


def _proc_node(nl, p0, p1, cnt, pn):
    def fn(nv, a0, a1, cv, w1a, w1b, b1, w2, b2, g, beta):
        agg = (a0 + a1) / jnp.maximum(cv, 1.0)
        h = _silu(_mm(nv, w1a) + _mm(agg, w1b) + b1)
        return nv + _ln(_mm(h, w2) + b2, g, beta)

    return _rowwise(fn, [nl, p0, p1, cnt], _node2_weights(pn), [H])[0]


def _dec_node(n, p0, p1, cnt, pn, po):
    dout = po["W2"].shape[1]

    def fn(nv, a0, a1, cv, w1a, w1b, b1, w2, b2, g, beta,
           w1o, b1o, w2o, b2o):
        agg = (a0 + a1) / jnp.maximum(cv, 1.0)
        h = _silu(_mm(nv, w1a) + _mm(agg, w1b) + b1)
        nn2 = _ln(_mm(h, w2) + b2, g, beta)
        h2 = jax.nn.sigmoid(_mm(nn2, w1o) + b1o)
        return _mm(h2, w2o) + b2o

    ws = _node2_weights(pn) + [po["W1"], po["b1"].reshape(1, -1),
                               po["W2"], po["b2"].reshape(1, -1)]
    return _rowwise(fn, [n, p0, p1, cnt], ws, [dout])[0]


def _sc_mesh():
    return plsc.VectorSubcoreMesh(core_axis_name="c", subcore_axis_name="s")


def _sc_gather_pair(t1, i1_3d, t2, i2_3d):
    nch = i1_3d.shape[1]
    rows = nch * CH
    E = NW * rows
    i1f = i1_3d.reshape(-1)
    i2f = i2_3d.reshape(-1)

    scratch = [pltpu.VMEM((CH,), I32), pltpu.VMEM((CH,), I32),
               pltpu.VMEM((CH, H), F32), pltpu.VMEM((CH, H), F32)]
    scratch += [pltpu.SemaphoreType.DMA] * 2

    @functools.partial(
        pl.kernel, mesh=_sc_mesh(),
        out_type=(jax.ShapeDtypeStruct((E, H), F32),
                  jax.ShapeDtypeStruct((E, H), F32)),
        scratch_types=scratch,
    )
    def k(t1r, i1r, t2r, i2r, o1r, o2r, iv1, iv2, rb1, rb2, sm1, sm2):
        wid = lax.axis_index("s") * 2 + lax.axis_index("c")

        def body(ci, carry):
            base = wid * rows + ci * CH
            pltpu.sync_copy(i1r.at[pl.ds(base, CH)], iv1)
            pltpu.sync_copy(i2r.at[pl.ds(base, CH)], iv2)
            c1 = pltpu.async_copy(t1r.at[iv1], rb1, sm1)
            c2 = pltpu.async_copy(t2r.at[iv2], rb2, sm2)
            c1.wait()
            c2.wait()
            pltpu.sync_copy(rb1, o1r.at[pl.ds(base, CH)])
            pltpu.sync_copy(rb2, o2r.at[pl.ds(base, CH)])
            return carry

        lax.fori_loop(0, nch, body, 0)

    return k(t1, i1f, t2, i2f)


def _sc_scatter(vals, idx_flat, npad):
    E = idx_flat.shape[0]
    rows = E // NW
    nch = rows // CH
    assert vals.shape[0] == E and npad % 128 == 0, (vals.shape, E, npad)
    G2 = nch // 2
    tail = nch % 2 == 1
    zr = npad // 16
    zrows = jnp.zeros((zr, H), F32)
    ones = jnp.ones((CH, H), F32)

    scratch = [pltpu.VMEM((CH, H), F32) for _ in range(2)]
    scratch += [pltpu.VMEM((CH,), I32) for _ in range(2)]
    scratch += [pltpu.VMEM((CH, H), F32)]
    scratch += [pltpu.VMEM_SHARED((npad, H), F32)]
    scratch += [pltpu.SemaphoreType.DMA] * 4

    @functools.partial(
        pl.kernel, mesh=_sc_mesh(),
        out_type=(jax.ShapeDtypeStruct((2, npad, H), F32),
                  jax.ShapeDtypeStruct((2, npad, H), F32)),
        scratch_types=scratch,
    )
    def k(valsr, idxr, zrr, onesr, sumsr, cntr, *rest):
        vb = rest[0:2]
        ivc = rest[2:4]
        ob = rest[4]
        acc = rest[5]
        smv = rest[6:8]
        sma = rest[8:10]
        cid = lax.axis_index("c")
        sid = lax.axis_index("s")
        wid = sid * 2 + cid
        base0 = wid * rows

        def zero_acc():
            pltpu.sync_copy(zrr, acc.at[pl.ds(sid * zr, zr)])

        def fire_v(p, c, with_vals):
            off = base0 + c * CH
            if with_vals:
                pltpu.async_copy(valsr.at[pl.ds(off, CH)], vb[p], smv[p])
            pltpu.async_copy(idxr.at[pl.ds(off, CH)], ivc[p], smv[p])

        def wait_v(p, with_vals):
            if with_vals:
                pltpu.make_async_copy(valsr.at[pl.ds(base0, CH)], vb[p],
                                      smv[p]).wait()
            pltpu.make_async_copy(idxr.at[pl.ds(base0, CH)], ivc[p],
                                  smv[p]).wait()

        def fire_a(p, src):
            pltpu.async_copy(src, acc.at[ivc[p]], sma[p], add=True)

        def wait_a(p, src):
            pltpu.make_async_copy(src, acc.at[ivc[p]], sma[p]).wait()

        def phase(with_vals, outr):
            zero_acc()
            plsc.subcore_barrier()
            src = (lambda p: vb[p]) if with_vals else (lambda p: ob)
            fire_v(0, 0, with_vals)

            def body(i, carry):
                a = 2 * i
                wait_v(0, with_vals)
                fire_v(1, a + 1, with_vals)
                fire_a(0, src(0))
                wait_v(1, with_vals)
                wait_a(0, src(0))

                @pl.when(i + 1 < G2)
                def _():
                    fire_v(0, a + 2, with_vals)

                fire_a(1, src(1))
                wait_a(1, src(1))
                return carry

            lax.fori_loop(0, G2, body, 0)
            if tail:
                c = nch - 1
                fire_v(0, c, with_vals)
                wait_v(0, with_vals)
                fire_a(0, src(0))
                wait_a(0, src(0))
            plsc.subcore_barrier()
            pltpu.sync_copy(acc.at[pl.ds(sid * zr, zr)],
                            outr.at[cid, pl.ds(sid * zr, zr)])

        pltpu.sync_copy(onesr, ob)
        phase(True, sumsr)
        plsc.subcore_barrier()
        phase(False, cntr)

    return k(vals, idx_flat, zrows, ones)


def _unpack_counts(cntp, n):
    return cntp[0, :n, :1] + cntp[1, :n, :1]


def _pad_to(x, n, value):
    if x.shape[0] == n:
        return x
    pad = [(0, n - x.shape[0])] + [(0, 0)] * (x.ndim - 1)
    return jnp.pad(x, pad, constant_values=value)


def _idx3(col, ep, fill=None):
    e = col.shape[0]
    if e != ep:
        if fill is None:
            col = jnp.concatenate(
                [col, (jnp.arange(ep - e, dtype=col.dtype) % _NG)])
        else:
            col = _pad_to(col, ep, fill)
    return col.reshape(NW, ep // (NW * CH), CH)


def kernel(pc2g_edge_idx, pc2g_edge_features, pc2g_node_features,
           g2g_edge_idx, g2g_edge_features, g2pc_edge_idx, g2pc_edge_features,
           params):
    NPC = pc2g_node_features.shape[0]
    NG = _NG
    NGP = ((NG + 1 + 127) // 128) * 128
    NPCP = ((NPC + 1 + 127) // 128) * 128
    E1 = pc2g_edge_idx.shape[0]
    E2 = g2g_edge_idx.shape[0]
    E3 = g2pc_edge_idx.shape[0]
    E1P = ((E1 + _GRAN - 1) // _GRAN) * _GRAN
    E2P = ((E2 + _GRAN - 1) // _GRAN) * _GRAN
    E3P = ((E3 + _GRAN - 1) // _GRAN) * _GRAN
    gmod = globals()
    gather_pair = gmod["_sc_gather_pair"]
    scatter = gmod["_sc_scatter"]

    enc, proc, dec = params["enc"], params["proc"], params["dec"]

    e1 = _ffb(_pad_to(pc2g_edge_features, E1P, 0.0), enc["embed_edge"])
    n_full = _ffb(_pad_to(pc2g_node_features, NPCP, 0.0), enc["embed_node"])
    s1g = _idx3(pc2g_edge_idx[:, 0], E1P)
    r1g = _idx3(pc2g_edge_idx[:, 1], E1P)
    r1f = _pad_to(pc2g_edge_idx[:, 1], E1P, NG)
    ns, nr = gather_pair(n_full, s1g, n_full, r1g)
    e2 = _edge_mlp(ns, nr, e1, enc["gn_edge"])
    sums, cntp1 = scatter(e2, r1f, NGP)
    n_lat = _enc_node(n_full[:NGP], sums[0], sums[1],
                      _unpack_counts(cntp1, NGP),
                      enc["gn_node"], enc["out"])

    pe = _ffb(_pad_to(g2g_edge_features, E2P, 0.0), proc["embed_edge"])
    s2g = _idx3(g2g_edge_idx[:, 0], E2P)
    r2g = _idx3(g2g_edge_idx[:, 1], E2P)
    r2f = _pad_to(g2g_edge_idx[:, 1], E2P, NG)

    def round_fn(carry, gp):
        nl, pev = carry
        ns2, nr2 = gather_pair(nl, s2g, nl, r2g)
        ne, pev = _proc_edge_mlp(ns2, nr2, pev, gp["edge"])
        sums2, cntp2 = scatter(ne, r2f, NGP)
        nl = _proc_node(nl, sums2[0], sums2[1],
                        _unpack_counts(cntp2, NGP), gp["node"])
        return (nl, pev), None

    stacked = jax.tree.map(lambda *xs: jnp.stack(xs), *proc["gn"])
    (n_lat, pe), _ = lax.scan(round_fn, (n_lat, pe), stacked)
    n_lat = _ffb(n_lat, proc["out"])

    de = _ffb(_pad_to(g2pc_edge_features, E3P, 0.0), dec["embed_edge"])
    s3g = _idx3(g2pc_edge_idx[:, 0], E3P)
    r3g = _idx3(g2pc_edge_idx[:, 1], E3P)
    r3f = _pad_to(g2pc_edge_idx[:, 1], E3P, NPC)
    ns3, nr3 = gather_pair(n_lat, s3g, n_full, r3g)
    de2 = _edge_mlp(ns3, nr3, de, dec["proc_edge"])
    sums3, cntp3 = scatter(de2, r3f, NPCP)
    out = _dec_node(n_full[:NPC], sums3[0, :NPC], sums3[1, :NPC],
                    _unpack_counts(cntp3, NPC),
                    dec["proc_node"], dec["out"])
    return out


# speedup vs baseline: 1.6775x; 1.0476x over previous
import functools

import jax
import jax.numpy as jnp
from jax import lax
from jax.experimental import pallas as pl
from jax.experimental.pallas import tpu as pltpu
from jax.experimental.pallas import tpu_sc as plsc

F32 = jnp.float32
I32 = jnp.int32
H = 128
_NG = 2500
NW = 32
CH = 80
K = 2
_GRAN = NW * CH


def _silu(x):
    return x * jax.nn.sigmoid(x)


def _ln(h, g, b):
    mu = jnp.mean(h, axis=-1, keepdims=True)
    var = jnp.mean((h - mu) ** 2, axis=-1, keepdims=True)
    return (h - mu) * lax.rsqrt(var + 1e-5) * g + b


def _pick_rows(n, cap=2048):
    best = None
    for cand in range(8, min(n, cap) + 1, 8):
        if n % cand == 0:
            best = cand
    return best if best is not None else n


def _mm(a, b):
    return jnp.dot(a, b, preferred_element_type=F32)


def _rowwise(fn, xs, ws, out_widths, R=None):
    E = xs[0].shape[0]
    if R is None:
        R = _pick_rows(E)
    grid = (E // R,)
    nx, nw = len(xs), len(ws)
    in_specs = [pl.BlockSpec((R, x.shape[1]), lambda i: (i, 0)) for x in xs]
    in_specs += [pl.BlockSpec(w.shape, lambda i: (0, 0)) for w in ws]
    out_specs = [pl.BlockSpec((R, wd), lambda i: (i, 0)) for wd in out_widths]
    out_shape = [jax.ShapeDtypeStruct((E, wd), F32) for wd in out_widths]

    def body(*refs):
        xr = [r[...] for r in refs[:nx]]
        wr = [r[...] for r in refs[nx:nx + nw]]
        outs = fn(*xr, *wr)
        if not isinstance(outs, tuple):
            outs = (outs,)
        for o_ref, o in zip(refs[nx + nw:], outs):
            o_ref[...] = o

    return pl.pallas_call(body, grid=grid, in_specs=in_specs,
                          out_specs=out_specs, out_shape=out_shape)(*xs, *ws)


def _ffb_weights(p):
    ws = [p["W1"], p["b1"].reshape(1, -1), p["W2"], p["b2"].reshape(1, -1)]
    if "g" in p:
        ws += [p["g"].reshape(1, -1), p["beta"].reshape(1, -1)]
    return ws


def _ffb(x, p, act=_silu):
    has_ln = "g" in p

    def fn(xv, w1, b1, w2, b2, *rest):
        h = act(_mm(xv, w1) + b1)
        h = _mm(h, w2) + b2
        if has_ln:
            h = _ln(h, rest[0], rest[1])
        return h

    return _rowwise(fn, [x], _ffb_weights(p), [p["W2"].shape[1]])[0]


def _edge_weights(p):
    w1 = p["W1"]
    return [w1[:H], w1[H:2 * H], w1[2 * H:], p["b1"].reshape(1, -1),
            p["W2"], p["b2"].reshape(1, -1),
            p["g"].reshape(1, -1), p["beta"].reshape(1, -1)]


def _edge_mlp(ns, nr, ef, p):
    def fn(a, b, c, w1a, w1b, w1c, b1, w2, b2, g, beta):
        h = _silu(_mm(a, w1a) + _mm(b, w1b) + _mm(c, w1c) + b1)
        return _ln(_mm(h, w2) + b2, g, beta)

    return _rowwise(fn, [ns, nr, ef], _edge_weights(p), [H])[0]


def _proc_edge_mlp(ns, nr, pe, p):
    def fn(a, b, c, w1a, w1b, w1c, b1, w2, b2, g, beta):
        h = _silu(_mm(a, w1a) + _mm(b, w1b) + _mm(c, w1c) + b1)
        ne = _ln(_mm(h, w2) + b2, g, beta)
        return ne, c + ne

    return _rowwise(fn, [ns, nr, pe], _edge_weights(p), [H, H])


def _node2_weights(p):
    w1 = p["W1"]
    return [w1[:H], w1[H:], p["b1"].reshape(1, -1), p["W2"],
            p["b2"].reshape(1, -1), p["g"].reshape(1, -1),
            p["beta"].reshape(1, -1)]


def _enc_node(n0, p0, p1, cnt, pn, po):
    def fn(nv, a0, a1, cv, w1a, w1b, b1, w2, b2, g, beta,
           w1o, b1o, w2o, b2o):
        agg = (a0 + a1) / jnp.maximum(cv, 1.0)
        h = _silu(_mm(nv, w1a) + _mm(agg, w1b) + b1)
        v = nv + _ln(_mm(h, w2) + b2, g, beta)
        h2 = _silu(_mm(v, w1o) + b1o)
        return _mm(h2, w2o) + b2o

    ws = _node2_weights(pn) + [po["W1"], po["b1"].reshape(1, -1),
                               po["W2"], po["b2"].reshape(1, -1)]
    return _rowwise(fn, [n0, p0, p1, cnt], ws, [H])[0]


def _proc_node(nl, p0, p1, cnt, pn):
    def fn(nv, a0, a1, cv, w1a, w1b, b1, w2, b2, g, beta):
        agg = (a0 + a1) / jnp.maximum(cv, 1.0)
        h = _silu(_mm(nv, w1a) + _mm(agg, w1b) + b1)
        return nv + _ln(_mm(h, w2) + b2, g, beta)

    return _rowwise(fn, [nl, p0, p1, cnt], _node2_weights(pn), [H])[0]


def _dec_node(n, p0, p1, cnt, pn, po):
    dout = po["W2"].shape[1]

    def fn(nv, a0, a1, cv, w1a, w1b, b1, w2, b2, g, beta,
           w1o, b1o, w2o, b2o):
        agg = (a0 + a1) / jnp.maximum(cv, 1.0)
        h = _silu(_mm(nv, w1a) + _mm(agg, w1b) + b1)
        nn2 = _ln(_mm(h, w2) + b2, g, beta)
        h2 = jax.nn.sigmoid(_mm(nn2, w1o) + b1o)
        return _mm(h2, w2o) + b2o

    ws = _node2_weights(pn) + [po["W1"], po["b1"].reshape(1, -1),
                               po["W2"], po["b2"].reshape(1, -1)]
    return _rowwise(fn, [n, p0, p1, cnt], ws, [dout])[0]


def _sc_mesh():
    return plsc.VectorSubcoreMesh(core_axis_name="c", subcore_axis_name="s")


def _sc_gather_pair(t1, i1_3d, t2, i2_3d):
    nch = i1_3d.shape[1]
    rows = nch * CH
    E = NW * rows
    G2 = nch // 2
    tail = nch % 2 == 1
    i1f = i1_3d.reshape(-1)
    i2f = i2_3d.reshape(-1)

    scratch = [pltpu.VMEM((CH,), I32) for _ in range(4)]
    scratch += [pltpu.VMEM((CH, H), F32) for _ in range(4)]
    scratch += [pltpu.SemaphoreType.DMA] * 6

    @functools.partial(
        pl.kernel, mesh=_sc_mesh(),
        out_type=(jax.ShapeDtypeStruct((E, H), F32),
                  jax.ShapeDtypeStruct((E, H), F32)),
        scratch_types=scratch,
    )
    def k(t1r, i1r, t2r, i2r, o1r, o2r, *rest):
        iv1 = rest[0:2]
        iv2 = rest[2:4]
        gb1 = rest[4:6]
        gb2 = rest[6:8]
        smv = rest[8:10]
        smg = rest[10:12]
        smw = rest[12:14]
        wid = lax.axis_index("s") * 2 + lax.axis_index("c")
        base0 = wid * rows

        def fire_i(p, c):
            off = base0 + c * CH
            pltpu.async_copy(i1r.at[pl.ds(off, CH)], iv1[p], smv[p])
            pltpu.async_copy(i2r.at[pl.ds(off, CH)], iv2[p], smv[p])

        def wait_i(p):
            pltpu.make_async_copy(i1r.at[pl.ds(base0, CH)], iv1[p],
                                  smv[p]).wait()
            pltpu.make_async_copy(i2r.at[pl.ds(base0, CH)], iv2[p],
                                  smv[p]).wait()

        def fire_g(p):
            pltpu.async_copy(t1r.at[iv1[p]], gb1[p], smg[p])
            pltpu.async_copy(t2r.at[iv2[p]], gb2[p], smg[p])

        def wait_g(p):
            pltpu.make_async_copy(t1r.at[iv1[p]], gb1[p], smg[p]).wait()
            pltpu.make_async_copy(t2r.at[iv2[p]], gb2[p], smg[p]).wait()

        def fire_w(p, c):
            off = base0 + c * CH
            pltpu.async_copy(gb1[p], o1r.at[pl.ds(off, CH)], smw[p])
            pltpu.async_copy(gb2[p], o2r.at[pl.ds(off, CH)], smw[p])

        def wait_w(p):
            pltpu.make_async_copy(gb1[p], o1r.at[pl.ds(base0, CH)],
                                  smw[p]).wait()
            pltpu.make_async_copy(gb2[p], o2r.at[pl.ds(base0, CH)],
                                  smw[p]).wait()

        fire_i(0, 0)

        def body(i, carry):
            a = 2 * i
            wait_i(0)
            fire_i(1, a + 1)
            fire_g(0)
            wait_g(0)
            fire_w(0, a)
            wait_i(1)

            @pl.when(i > 0)
            def _():
                wait_w(1)

            fire_g(1)

            @pl.when(i + 1 < G2)
            def _():
                fire_i(0, a + 2)

            wait_g(1)
            fire_w(1, a + 1)
            wait_w(0)
            return carry

        lax.fori_loop(0, G2, body, 0)
        wait_w(1)
        if tail:
            c = nch - 1
            fire_i(0, c)
            wait_i(0)
            fire_g(0)
            wait_g(0)
            fire_w(0, c)
            wait_w(0)

    return k(t1, i1f, t2, i2f)


def _sc_scatter(vals, idx_flat, npad):
    E = idx_flat.shape[0]
    rows = E // NW
    nch = rows // CH
    assert vals.shape[0] == E and npad % 128 == 0, (vals.shape, E, npad)
    G2 = nch // 2
    tail = nch % 2 == 1
    zr = npad // 16
    zrows = jnp.zeros((zr, H), F32)
    ones = jnp.ones((CH, H), F32)

    scratch = [pltpu.VMEM((CH, H), F32) for _ in range(2)]
    scratch += [pltpu.VMEM((CH,), I32) for _ in range(2)]
    scratch += [pltpu.VMEM((CH, H), F32)]
    scratch += [pltpu.VMEM_SHARED((npad, H), F32)]
    scratch += [pltpu.SemaphoreType.DMA] * 4

    @functools.partial(
        pl.kernel, mesh=_sc_mesh(),
        out_type=(jax.ShapeDtypeStruct((2, npad, H), F32),
                  jax.ShapeDtypeStruct((2, npad, H), F32)),
        scratch_types=scratch,
    )
    def k(valsr, idxr, zrr, onesr, sumsr, cntr, *rest):
        vb = rest[0:2]
        ivc = rest[2:4]
        ob = rest[4]
        acc = rest[5]
        smv = rest[6:8]
        sma = rest[8:10]
        cid = lax.axis_index("c")
        sid = lax.axis_index("s")
        wid = sid * 2 + cid
        base0 = wid * rows

        def zero_acc():
            pltpu.sync_copy(zrr, acc.at[pl.ds(sid * zr, zr)])

        def fire_v(p, c, with_vals):
            off = base0 + c * CH
            if with_vals:
                pltpu.async_copy(valsr.at[pl.ds(off, CH)], vb[p], smv[p])
            pltpu.async_copy(idxr.at[pl.ds(off, CH)], ivc[p], smv[p])

        def wait_v(p, with_vals):
            if with_vals:
                pltpu.make_async_copy(valsr.at[pl.ds(base0, CH)], vb[p],
                                      smv[p]).wait()
            pltpu.make_async_copy(idxr.at[pl.ds(base0, CH)], ivc[p],
                                  smv[p]).wait()

        def fire_a(p, src):
            pltpu.async_copy(src, acc.at[ivc[p]], sma[p], add=True)

        def wait_a(p, src):
            pltpu.make_async_copy(src, acc.at[ivc[p]], sma[p]).wait()

        def phase(with_vals, outr):
            zero_acc()
            plsc.subcore_barrier()
            src = (lambda p: vb[p]) if with_vals else (lambda p: ob)
            fire_v(0, 0, with_vals)

            def body(i, carry):
                a = 2 * i
                wait_v(0, with_vals)
                fire_v(1, a + 1, with_vals)
                fire_a(0, src(0))
                wait_v(1, with_vals)
                wait_a(0, src(0))

                @pl.when(i + 1 < G2)
                def _():
                    fire_v(0, a + 2, with_vals)

                fire_a(1, src(1))
                wait_a(1, src(1))
                return carry

            lax.fori_loop(0, G2, body, 0)
            if tail:
                c = nch - 1
                fire_v(0, c, with_vals)
                wait_v(0, with_vals)
                fire_a(0, src(0))
                wait_a(0, src(0))
            plsc.subcore_barrier()
            pltpu.sync_copy(acc.at[pl.ds(sid * zr, zr)],
                            outr.at[cid, pl.ds(sid * zr, zr)])

        pltpu.sync_copy(onesr, ob)
        phase(True, sumsr)
        plsc.subcore_barrier()
        phase(False, cntr)

    return k(vals, idx_flat, zrows, ones)


def _unpack_counts(cntp, n):
    return cntp[0, :n, :1] + cntp[1, :n, :1]


def _pad_to(x, n, value):
    if x.shape[0] == n:
        return x
    pad = [(0, n - x.shape[0])] + [(0, 0)] * (x.ndim - 1)
    return jnp.pad(x, pad, constant_values=value)


def _idx3(col, ep, fill=None):
    e = col.shape[0]
    if e != ep:
        if fill is None:
            col = jnp.concatenate(
                [col, (jnp.arange(ep - e, dtype=col.dtype) % _NG)])
        else:
            col = _pad_to(col, ep, fill)
    return col.reshape(NW, ep // (NW * CH), CH)


def kernel(pc2g_edge_idx, pc2g_edge_features, pc2g_node_features,
           g2g_edge_idx, g2g_edge_features, g2pc_edge_idx, g2pc_edge_features,
           params):
    NPC = pc2g_node_features.shape[0]
    NG = _NG
    NGP = ((NG + 1 + 127) // 128) * 128
    NPCP = ((NPC + 1 + 127) // 128) * 128
    E1 = pc2g_edge_idx.shape[0]
    E2 = g2g_edge_idx.shape[0]
    E3 = g2pc_edge_idx.shape[0]
    E1P = ((E1 + _GRAN - 1) // _GRAN) * _GRAN
    E2P = ((E2 + _GRAN - 1) // _GRAN) * _GRAN
    E3P = ((E3 + _GRAN - 1) // _GRAN) * _GRAN
    gmod = globals()
    gather_pair = gmod["_sc_gather_pair"]
    scatter = gmod["_sc_scatter"]

    enc, proc, dec = params["enc"], params["proc"], params["dec"]

    e1 = _ffb(_pad_to(pc2g_edge_features, E1P, 0.0), enc["embed_edge"])
    n_full = _ffb(_pad_to(pc2g_node_features, NPCP, 0.0), enc["embed_node"])
    s1g = _idx3(pc2g_edge_idx[:, 0], E1P)
    r1g = _idx3(pc2g_edge_idx[:, 1], E1P)
    r1f = _pad_to(pc2g_edge_idx[:, 1], E1P, NG)
    ns, nr = gather_pair(n_full, s1g, n_full, r1g)
    e2 = _edge_mlp(ns, nr, e1, enc["gn_edge"])
    sums, cntp1 = scatter(e2, r1f, NGP)
    n_lat = _enc_node(n_full[:NGP], sums[0], sums[1],
                      _unpack_counts(cntp1, NGP),
                      enc["gn_node"], enc["out"])

    pe = _ffb(_pad_to(g2g_edge_features, E2P, 0.0), proc["embed_edge"])
    s2g = _idx3(g2g_edge_idx[:, 0], E2P)
    r2g = _idx3(g2g_edge_idx[:, 1], E2P)
    r2f = _pad_to(g2g_edge_idx[:, 1], E2P, NG)

    def round_fn(carry, gp):
        nl, pev = carry
        ns2, nr2 = gather_pair(nl, s2g, nl, r2g)
        ne, pev = _proc_edge_mlp(ns2, nr2, pev, gp["edge"])
        sums2, cntp2 = scatter(ne, r2f, NGP)
        nl = _proc_node(nl, sums2[0], sums2[1],
                        _unpack_counts(cntp2, NGP), gp["node"])
        return (nl, pev), None

    stacked = jax.tree.map(lambda *xs: jnp.stack(xs), *proc["gn"])
    (n_lat, pe), _ = lax.scan(round_fn, (n_lat, pe), stacked)
    n_lat = _ffb(n_lat, proc["out"])

    de = _ffb(_pad_to(g2pc_edge_features, E3P, 0.0), dec["embed_edge"])
    s3g = _idx3(g2pc_edge_idx[:, 0], E3P)
    r3g = _idx3(g2pc_edge_idx[:, 1], E3P)
    r3f = _pad_to(g2pc_edge_idx[:, 1], E3P, NPC)
    ns3, nr3 = gather_pair(n_lat, s3g, n_full, r3g)
    de2 = _edge_mlp(ns3, nr3, de, dec["proc_edge"])
    sums3, cntp3 = scatter(de2, r3f, NPCP)
    out = _dec_node(n_full[:NPC], sums3[0, :NPC], sums3[1, :NPC],
                    _unpack_counts(cntp3, NPC),
                    dec["proc_node"], dec["out"])
    return out


# speedup vs baseline: 1.8226x; 1.0865x over previous
import functools

import jax
import jax.numpy as jnp
from jax import lax
from jax.experimental import pallas as pl
from jax.experimental.pallas import tpu as pltpu
from jax.experimental.pallas import tpu_sc as plsc

F32 = jnp.float32
I32 = jnp.int32
H = 128
_NG = 2500
NW = 32
CH = 80
K = 2
_GRAN = NW * CH


def _silu(x):
    return x * jax.nn.sigmoid(x)


def _ln(h, g, b):
    mu = jnp.mean(h, axis=-1, keepdims=True)
    var = jnp.mean((h - mu) ** 2, axis=-1, keepdims=True)
    return (h - mu) * lax.rsqrt(var + 1e-5) * g + b


def _pick_rows(n, cap=2048):
    best = None
    for cand in range(8, min(n, cap) + 1, 8):
        if n % cand == 0:
            best = cand
    return best if best is not None else n


def _mm(a, b):
    return jnp.dot(a, b, preferred_element_type=F32)


def _rowwise(fn, xs, ws, out_widths, R=None):
    E = xs[0].shape[0]
    if R is None:
        R = _pick_rows(E)
    grid = (E // R,)
    nx, nw = len(xs), len(ws)
    in_specs = [pl.BlockSpec((R, x.shape[1]), lambda i: (i, 0)) for x in xs]
    in_specs += [pl.BlockSpec(w.shape, lambda i: (0, 0)) for w in ws]
    out_specs = [pl.BlockSpec((R, wd), lambda i: (i, 0)) for wd in out_widths]
    out_shape = [jax.ShapeDtypeStruct((E, wd), F32) for wd in out_widths]

    def body(*refs):
        xr = [r[...] for r in refs[:nx]]
        wr = [r[...] for r in refs[nx:nx + nw]]
        outs = fn(*xr, *wr)
        if not isinstance(outs, tuple):
            outs = (outs,)
        for o_ref, o in zip(refs[nx + nw:], outs):
            o_ref[...] = o

    return pl.pallas_call(body, grid=grid, in_specs=in_specs,
                          out_specs=out_specs, out_shape=out_shape)(*xs, *ws)


def _ffb_weights(p):
    ws = [p["W1"], p["b1"].reshape(1, -1), p["W2"], p["b2"].reshape(1, -1)]
    if "g" in p:
        ws += [p["g"].reshape(1, -1), p["beta"].reshape(1, -1)]
    return ws


def _ffb(x, p, act=_silu):
    has_ln = "g" in p

    def fn(xv, w1, b1, w2, b2, *rest):
        h = act(_mm(xv, w1) + b1)
        h = _mm(h, w2) + b2
        if has_ln:
            h = _ln(h, rest[0], rest[1])
        return h

    return _rowwise(fn, [x], _ffb_weights(p), [p["W2"].shape[1]])[0]


def _edge_weights(p):
    w1 = p["W1"]
    return [w1[:H], w1[H:2 * H], w1[2 * H:], p["b1"].reshape(1, -1),
            p["W2"], p["b2"].reshape(1, -1),
            p["g"].reshape(1, -1), p["beta"].reshape(1, -1)]


def _edge_mlp_embed(ns, nr, ef_raw, pe_, p):
    def fn(a, b, x, ew1, eb1, ew2, eb2, eg, ebeta,
           w1a, w1b, w1c, b1, w2, b2, g, beta):
        e = _ln(_mm(_silu(_mm(x, ew1) + eb1), ew2) + eb2, eg, ebeta)
        h = _silu(_mm(a, w1a) + _mm(b, w1b) + _mm(e, w1c) + b1)
        return _ln(_mm(h, w2) + b2, g, beta)

    ws = _ffb_weights(pe_) + _edge_weights(p)
    return _rowwise(fn, [ns, nr, ef_raw], ws, [H])[0]


def _proc_edge_mlp(ns, nr, pe, p):
    def fn(a, b, c, w1a, w1b, w1c, b1, w2, b2, g, beta):
        h = _silu(_mm(a, w1a) + _mm(b, w1b) + _mm(c, w1c) + b1)
        ne = _ln(_mm(h, w2) + b2, g, beta)
        return ne, c + ne

    return _rowwise(fn, [ns, nr, pe], _edge_weights(p), [H, H])


def _node2_weights(p):
    w1 = p["W1"]
    return [w1[:H], w1[H:], p["b1"].reshape(1, -1), p["W2"],
            p["b2"].reshape(1, -1), p["g"].reshape(1, -1),
            p["beta"].reshape(1, -1)]


def _enc_node(n0, p0, p1, cnt, pn, po):
    def fn(nv, a0, a1, cv, w1a, w1b, b1, w2, b2, g, beta,
           w1o, b1o, w2o, b2o):
        agg = (a0 + a1) / jnp.maximum(cv, 1.0)
        h = _silu(_mm(nv, w1a) + _mm(agg, w1b) + b1)
        v = nv + _ln(_mm(h, w2) + b2, g, beta)
        h2 = _silu(_mm(v, w1o) + b1o)
        return _mm(h2, w2o) + b2o

    ws = _node2_weights(pn) + [po["W1"], po["b1"].reshape(1, -1),
                               po["W2"], po["b2"].reshape(1, -1)]
    return _rowwise(fn, [n0, p0, p1, cnt], ws, [H])[0]


def _proc_node(nl, p0, p1, cnt, pn):
    def fn(nv, a0, a1, cv, w1a, w1b, b1, w2, b2, g, beta):
        agg = (a0 + a1) / jnp.maximum(cv, 1.0)
        h = _silu(_mm(nv, w1a) + _mm(agg, w1b) + b1)
        return nv + _ln(_mm(h, w2) + b2, g, beta)

    return _rowwise(fn, [nl, p0, p1, cnt], _node2_weights(pn), [H])[0]


def _dec_node(n, p0, p1, cnt, pn, po):
    dout = po["W2"].shape[1]

    def fn(nv, a0, a1, cv, w1a, w1b, b1, w2, b2, g, beta,
           w1o, b1o, w2o, b2o):
        agg = (a0 + a1) / jnp.maximum(cv, 1.0)
        h = _silu(_mm(nv, w1a) + _mm(agg, w1b) + b1)
        nn2 = _ln(_mm(h, w2) + b2, g, beta)
        h2 = jax.nn.sigmoid(_mm(nn2, w1o) + b1o)
        return _mm(h2, w2o) + b2o

    ws = _node2_weights(pn) + [po["W1"], po["b1"].reshape(1, -1),
                               po["W2"], po["b2"].reshape(1, -1)]
    return _rowwise(fn, [n, p0, p1, cnt], ws, [dout])[0]


def _sc_mesh():
    return plsc.VectorSubcoreMesh(core_axis_name="c", subcore_axis_name="s")


def _sc_gather_pair(t1, i1_3d, t2, i2_3d):
    nch = i1_3d.shape[1]
    rows = nch * CH
    E = NW * rows
    G2 = nch // 2
    tail = nch % 2 == 1
    i1f = i1_3d.reshape(-1)
    i2f = i2_3d.reshape(-1)

    scratch = [pltpu.VMEM((CH,), I32) for _ in range(4)]
    scratch += [pltpu.VMEM((CH, H), F32) for _ in range(4)]
    scratch += [pltpu.SemaphoreType.DMA] * 6

    @functools.partial(
        pl.kernel, mesh=_sc_mesh(),
        out_type=(jax.ShapeDtypeStruct((E, H), F32),
                  jax.ShapeDtypeStruct((E, H), F32)),
        scratch_types=scratch,
    )
    def k(t1r, i1r, t2r, i2r, o1r, o2r, *rest):
        iv1 = rest[0:2]
        iv2 = rest[2:4]
        gb1 = rest[4:6]
        gb2 = rest[6:8]
        smv = rest[8:10]
        smg = rest[10:12]
        smw = rest[12:14]
        wid = lax.axis_index("s") * 2 + lax.axis_index("c")
        base0 = wid * rows

        def fire_i(p, c):
            off = base0 + c * CH
            pltpu.async_copy(i1r.at[pl.ds(off, CH)], iv1[p], smv[p])
            pltpu.async_copy(i2r.at[pl.ds(off, CH)], iv2[p], smv[p])

        def wait_i(p):
            pltpu.make_async_copy(i1r.at[pl.ds(base0, CH)], iv1[p],
                                  smv[p]).wait()
            pltpu.make_async_copy(i2r.at[pl.ds(base0, CH)], iv2[p],
                                  smv[p]).wait()

        def fire_g(p):
            pltpu.async_copy(t1r.at[iv1[p]], gb1[p], smg[p])
            pltpu.async_copy(t2r.at[iv2[p]], gb2[p], smg[p])

        def wait_g(p):
            pltpu.make_async_copy(t1r.at[iv1[p]], gb1[p], smg[p]).wait()
            pltpu.make_async_copy(t2r.at[iv2[p]], gb2[p], smg[p]).wait()

        def fire_w(p, c):
            off = base0 + c * CH
            pltpu.async_copy(gb1[p], o1r.at[pl.ds(off, CH)], smw[p])
            pltpu.async_copy(gb2[p], o2r.at[pl.ds(off, CH)], smw[p])

        def wait_w(p):
            pltpu.make_async_copy(gb1[p], o1r.at[pl.ds(base0, CH)],
                                  smw[p]).wait()
            pltpu.make_async_copy(gb2[p], o2r.at[pl.ds(base0, CH)],
                                  smw[p]).wait()

        fire_i(0, 0)

        def body(i, carry):
            a = 2 * i
            wait_i(0)
            fire_i(1, a + 1)
            fire_g(0)
            wait_g(0)
            fire_w(0, a)
            wait_i(1)

            @pl.when(i > 0)
            def _():
                wait_w(1)

            fire_g(1)

            @pl.when(i + 1 < G2)
            def _():
                fire_i(0, a + 2)

            wait_g(1)
            fire_w(1, a + 1)
            wait_w(0)
            return carry

        lax.fori_loop(0, G2, body, 0)
        wait_w(1)
        if tail:
            c = nch - 1
            fire_i(0, c)
            wait_i(0)
            fire_g(0)
            wait_g(0)
            fire_w(0, c)
            wait_w(0)

    return k(t1, i1f, t2, i2f)


def _sc_scatter(vals, idx_flat, npad):
    E = idx_flat.shape[0]
    rows = E // NW
    nch = rows // CH
    assert vals.shape[0] == E and npad % 128 == 0, (vals.shape, E, npad)
    G2 = nch // 2
    tail = nch % 2 == 1
    zr = npad // 16
    zrows = jnp.zeros((zr, H), F32)
    ones = jnp.ones((CH, H), F32)

    scratch = [pltpu.VMEM((CH, H), F32) for _ in range(2)]
    scratch += [pltpu.VMEM((CH,), I32) for _ in range(2)]
    scratch += [pltpu.VMEM((CH, H), F32)]
    scratch += [pltpu.VMEM_SHARED((npad, H), F32)]
    scratch += [pltpu.SemaphoreType.DMA] * 4

    @functools.partial(
        pl.kernel, mesh=_sc_mesh(),
        out_type=(jax.ShapeDtypeStruct((2, npad, H), F32),
                  jax.ShapeDtypeStruct((2, npad, H), F32)),
        scratch_types=scratch,
    )
    def k(valsr, idxr, zrr, onesr, sumsr, cntr, *rest):
        vb = rest[0:2]
        ivc = rest[2:4]
        ob = rest[4]
        acc = rest[5]
        smv = rest[6:8]
        sma = rest[8:10]
        cid = lax.axis_index("c")
        sid = lax.axis_index("s")
        wid = sid * 2 + cid
        base0 = wid * rows

        def zero_acc():
            pltpu.sync_copy(zrr, acc.at[pl.ds(sid * zr, zr)])

        def fire_v(p, c, with_vals):
            off = base0 + c * CH
            if with_vals:
                pltpu.async_copy(valsr.at[pl.ds(off, CH)], vb[p], smv[p])
            pltpu.async_copy(idxr.at[pl.ds(off, CH)], ivc[p], smv[p])

        def wait_v(p, with_vals):
            if with_vals:
                pltpu.make_async_copy(valsr.at[pl.ds(base0, CH)], vb[p],
                                      smv[p]).wait()
            pltpu.make_async_copy(idxr.at[pl.ds(base0, CH)], ivc[p],
                                  smv[p]).wait()

        def fire_a(p, src):
            pltpu.async_copy(src, acc.at[ivc[p]], sma[p], add=True)

        def wait_a(p, src):
            pltpu.make_async_copy(src, acc.at[ivc[p]], sma[p]).wait()

        def phase(with_vals, outr):
            zero_acc()
            plsc.subcore_barrier()
            src = (lambda p: vb[p]) if with_vals else (lambda p: ob)
            fire_v(0, 0, with_vals)

            def body(i, carry):
                a = 2 * i
                wait_v(0, with_vals)
                fire_v(1, a + 1, with_vals)
                fire_a(0, src(0))
                wait_v(1, with_vals)
                wait_a(0, src(0))

                @pl.when(i + 1 < G2)
                def _():
                    fire_v(0, a + 2, with_vals)

                fire_a(1, src(1))
                wait_a(1, src(1))
                return carry

            lax.fori_loop(0, G2, body, 0)
            if tail:
                c = nch - 1
                fire_v(0, c, with_vals)
                wait_v(0, with_vals)
                fire_a(0, src(0))
                wait_a(0, src(0))
            plsc.subcore_barrier()
            pltpu.sync_copy(acc.at[pl.ds(sid * zr, zr)],
                            outr.at[cid, pl.ds(sid * zr, zr)])

        pltpu.sync_copy(onesr, ob)
        phase(True, sumsr)
        plsc.subcore_barrier()
        phase(False, cntr)

    return k(vals, idx_flat, zrows, ones)


def _unpack_counts(cntp, n):
    return cntp[0, :n, :1] + cntp[1, :n, :1]


def _pad_to(x, n, value):
    if x.shape[0] == n:
        return x
    pad = [(0, n - x.shape[0])] + [(0, 0)] * (x.ndim - 1)
    return jnp.pad(x, pad, constant_values=value)


def _idx3(col, ep, fill=None):
    e = col.shape[0]
    if e != ep:
        if fill is None:
            col = jnp.concatenate(
                [col, (jnp.arange(ep - e, dtype=col.dtype) % _NG)])
        else:
            col = _pad_to(col, ep, fill)
    return col.reshape(NW, ep // (NW * CH), CH)


def kernel(pc2g_edge_idx, pc2g_edge_features, pc2g_node_features,
           g2g_edge_idx, g2g_edge_features, g2pc_edge_idx, g2pc_edge_features,
           params):
    NPC = pc2g_node_features.shape[0]
    NG = _NG
    NGP = ((NG + 1 + 127) // 128) * 128
    NPCP = ((NPC + 1 + 127) // 128) * 128
    E1 = pc2g_edge_idx.shape[0]
    E2 = g2g_edge_idx.shape[0]
    E3 = g2pc_edge_idx.shape[0]
    E1P = ((E1 + _GRAN - 1) // _GRAN) * _GRAN
    E2P = ((E2 + _GRAN - 1) // _GRAN) * _GRAN
    E3P = ((E3 + _GRAN - 1) // _GRAN) * _GRAN
    gmod = globals()
    gather_pair = gmod["_sc_gather_pair"]
    scatter = gmod["_sc_scatter"]

    enc, proc, dec = params["enc"], params["proc"], params["dec"]

    ef1 = _pad_to(pc2g_edge_features, E1P, 0.0)
    n_full = _ffb(_pad_to(pc2g_node_features, NPCP, 0.0), enc["embed_node"])
    s1g = _idx3(pc2g_edge_idx[:, 0], E1P)
    r1g = _idx3(pc2g_edge_idx[:, 1], E1P)
    r1f = _pad_to(pc2g_edge_idx[:, 1], E1P, NG)
    ns, nr = gather_pair(n_full, s1g, n_full, r1g)
    e2 = _edge_mlp_embed(ns, nr, ef1, enc["embed_edge"], enc["gn_edge"])
    sums, cntp1 = scatter(e2, r1f, NGP)
    n_lat = _enc_node(n_full[:NGP], sums[0], sums[1],
                      _unpack_counts(cntp1, NGP),
                      enc["gn_node"], enc["out"])

    pe = _ffb(_pad_to(g2g_edge_features, E2P, 0.0), proc["embed_edge"])
    s2g = _idx3(g2g_edge_idx[:, 0], E2P)
    r2g = _idx3(g2g_edge_idx[:, 1], E2P)
    r2f = _pad_to(g2g_edge_idx[:, 1], E2P, NG)

    def round_fn(carry, gp):
        nl, pev = carry
        ns2, nr2 = gather_pair(nl, s2g, nl, r2g)
        ne, pev = _proc_edge_mlp(ns2, nr2, pev, gp["edge"])
        sums2, cntp2 = scatter(ne, r2f, NGP)
        nl = _proc_node(nl, sums2[0], sums2[1],
                        _unpack_counts(cntp2, NGP), gp["node"])
        return (nl, pev), None

    stacked = jax.tree.map(lambda *xs: jnp.stack(xs), *proc["gn"])
    (n_lat, pe), _ = lax.scan(round_fn, (n_lat, pe), stacked)
    n_lat = _ffb(n_lat, proc["out"])

    ef3 = _pad_to(g2pc_edge_features, E3P, 0.0)
    s3g = _idx3(g2pc_edge_idx[:, 0], E3P)
    r3g = _idx3(g2pc_edge_idx[:, 1], E3P)
    r3f = _pad_to(g2pc_edge_idx[:, 1], E3P, NPC)
    ns3, nr3 = gather_pair(n_lat, s3g, n_full, r3g)
    de2 = _edge_mlp_embed(ns3, nr3, ef3, dec["embed_edge"],
                          dec["proc_edge"])
    sums3, cntp3 = scatter(de2, r3f, NPCP)
    out = _dec_node(n_full[:NPC], sums3[0, :NPC], sums3[1, :NPC],
                    _unpack_counts(cntp3, NPC),
                    dec["proc_node"], dec["out"])
    return out


# speedup vs baseline: 1.8274x; 1.0026x over previous
import functools

import jax
import jax.numpy as jnp
from jax import lax
from jax.experimental import pallas as pl
from jax.experimental.pallas import tpu as pltpu
from jax.experimental.pallas import tpu_sc as plsc

F32 = jnp.float32
I32 = jnp.int32
H = 128
_NG = 2500
NW = 32
CH = 80
_GRAN = NW * CH


def _silu(x):
    return x * jax.nn.sigmoid(x)


def _ln(h, g, b):
    mu = jnp.mean(h, axis=-1, keepdims=True)
    var = jnp.mean((h - mu) ** 2, axis=-1, keepdims=True)
    return (h - mu) * lax.rsqrt(var + 1e-5) * g + b


def _pick_rows(n, cap=2048):
    best = None
    for cand in range(8, min(n, cap) + 1, 8):
        if n % cand == 0:
            best = cand
    return best if best is not None else n


def _mm(a, b):
    return jnp.dot(a, b, preferred_element_type=F32)


def _rowwise(fn, xs, ws, out_widths, R=None):
    E = xs[0].shape[0]
    if R is None:
        R = _pick_rows(E)
    grid = (E // R,)
    nx, nw = len(xs), len(ws)
    in_specs = [pl.BlockSpec((R, x.shape[1]), lambda i: (i, 0)) for x in xs]
    in_specs += [pl.BlockSpec(w.shape, lambda i: (0, 0)) for w in ws]
    out_specs = [pl.BlockSpec((R, wd), lambda i: (i, 0)) for wd in out_widths]
    out_shape = [jax.ShapeDtypeStruct((E, wd), F32) for wd in out_widths]

    def body(*refs):
        xr = [r[...] for r in refs[:nx]]
        wr = [r[...] for r in refs[nx:nx + nw]]
        outs = fn(*xr, *wr)
        if not isinstance(outs, tuple):
            outs = (outs,)
        for o_ref, o in zip(refs[nx + nw:], outs):
            o_ref[...] = o

    return pl.pallas_call(body, grid=grid, in_specs=in_specs,
                          out_specs=out_specs, out_shape=out_shape)(*xs, *ws)


def _ffb_weights(p):
    ws = [p["W1"], p["b1"].reshape(1, -1), p["W2"], p["b2"].reshape(1, -1)]
    if "g" in p:
        ws += [p["g"].reshape(1, -1), p["beta"].reshape(1, -1)]
    return ws


def _ffb(x, p, act=_silu):
    has_ln = "g" in p

    def fn(xv, w1, b1, w2, b2, *rest):
        h = act(_mm(xv, w1) + b1)
        h = _mm(h, w2) + b2
        if has_ln:
            h = _ln(h, rest[0], rest[1])
        return h

    return _rowwise(fn, [x], _ffb_weights(p), [p["W2"].shape[1]])[0]


def _edge_weights(p):
    w1 = p["W1"]
    return [w1[:H], w1[H:2 * H], w1[2 * H:], p["b1"].reshape(1, -1),
            p["W2"], p["b2"].reshape(1, -1),
            p["g"].reshape(1, -1), p["beta"].reshape(1, -1)]


def _edge_mlp_embed(ns, nr, ef_raw, pe_, p):
    def fn(a, b, x, ew1, eb1, ew2, eb2, eg, ebeta,
           w1a, w1b, w1c, b1, w2, b2, g, beta):
        e = _ln(_mm(_silu(_mm(x, ew1) + eb1), ew2) + eb2, eg, ebeta)
        h = _silu(_mm(a, w1a) + _mm(b, w1b) + _mm(e, w1c) + b1)
        return _ln(_mm(h, w2) + b2, g, beta)

    ws = _ffb_weights(pe_) + _edge_weights(p)
    return _rowwise(fn, [ns, nr, ef_raw], ws, [H])[0]


def _proc_edge_mlp(ns, nr, pe, p):
    def fn(a, b, c, w1a, w1b, w1c, b1, w2, b2, g, beta):
        h = _silu(_mm(a, w1a) + _mm(b, w1b) + _mm(c, w1c) + b1)
        ne = _ln(_mm(h, w2) + b2, g, beta)
        return ne, c + ne

    return _rowwise(fn, [ns, nr, pe], _edge_weights(p), [H, H])


def _node2_weights(p):
    w1 = p["W1"]
    return [w1[:H], w1[H:], p["b1"].reshape(1, -1), p["W2"],
            p["b2"].reshape(1, -1), p["g"].reshape(1, -1),
            p["beta"].reshape(1, -1)]


def _enc_node(n0, p0, p1, cnt, pn, po):
    def fn(nv, a0, a1, cv, w1a, w1b, b1, w2, b2, g, beta,
           w1o, b1o, w2o, b2o):
        agg = (a0 + a1) / jnp.maximum(cv, 1.0)
        h = _silu(_mm(nv, w1a) + _mm(agg, w1b) + b1)
        v = nv + _ln(_mm(h, w2) + b2, g, beta)
        h2 = _silu(_mm(v, w1o) + b1o)
        return _mm(h2, w2o) + b2o

    ws = _node2_weights(pn) + [po["W1"], po["b1"].reshape(1, -1),
                               po["W2"], po["b2"].reshape(1, -1)]
    return _rowwise(fn, [n0, p0, p1, cnt], ws, [H])[0]


def _proc_node(nl, p0, p1, cnt, pn):
    def fn(nv, a0, a1, cv, w1a, w1b, b1, w2, b2, g, beta):
        agg = (a0 + a1) / jnp.maximum(cv, 1.0)
        h = _silu(_mm(nv, w1a) + _mm(agg, w1b) + b1)
        return nv + _ln(_mm(h, w2) + b2, g, beta)

    return _rowwise(fn, [nl, p0, p1, cnt], _node2_weights(pn), [H])[0]


def _dec_node(n, p0, p1, cnt, pn, po):
    dout = po["W2"].shape[1]

    def fn(nv, a0, a1, cv, w1a, w1b, b1, w2, b2, g, beta,
           w1o, b1o, w2o, b2o):
        agg = (a0 + a1) / jnp.maximum(cv, 1.0)
        h = _silu(_mm(nv, w1a) + _mm(agg, w1b) + b1)
        nn2 = _ln(_mm(h, w2) + b2, g, beta)
        h2 = jax.nn.sigmoid(_mm(nn2, w1o) + b1o)
        return _mm(h2, w2o) + b2o

    ws = _node2_weights(pn) + [po["W1"], po["b1"].reshape(1, -1),
                               po["W2"], po["b2"].reshape(1, -1)]
    return _rowwise(fn, [n, p0, p1, cnt], ws, [dout])[0]


def _sc_mesh():
    return plsc.VectorSubcoreMesh(core_axis_name="c", subcore_axis_name="s")


def _sc_gather_pair(t1, i1_3d, t2, i2_3d):
    nch = i1_3d.shape[1]
    rows = nch * CH
    E = NW * rows
    G2 = nch // 2
    tail = nch % 2 == 1
    i1f = i1_3d.reshape(-1)
    i2f = i2_3d.reshape(-1)

    scratch = [pltpu.VMEM((CH,), I32) for _ in range(4)]
    scratch += [pltpu.VMEM((CH, H), F32) for _ in range(4)]
    scratch += [pltpu.SemaphoreType.DMA] * 6

    @functools.partial(
        pl.kernel, mesh=_sc_mesh(),
        out_type=(jax.ShapeDtypeStruct((E, H), F32),
                  jax.ShapeDtypeStruct((E, H), F32)),
        scratch_types=scratch,
    )
    def k(t1r, i1r, t2r, i2r, o1r, o2r, *rest):
        iv1 = rest[0:2]
        iv2 = rest[2:4]
        gb1 = rest[4:6]
        gb2 = rest[6:8]
        smv = rest[8:10]
        smg = rest[10:12]
        smw = rest[12:14]
        wid = lax.axis_index("s") * 2 + lax.axis_index("c")
        base0 = wid * rows

        def fire_i(p, c):
            off = base0 + c * CH
            pltpu.async_copy(i1r.at[pl.ds(off, CH)], iv1[p], smv[p])
            pltpu.async_copy(i2r.at[pl.ds(off, CH)], iv2[p], smv[p])

        def wait_i(p):
            pltpu.make_async_copy(i1r.at[pl.ds(base0, CH)], iv1[p],
                                  smv[p]).wait()
            pltpu.make_async_copy(i2r.at[pl.ds(base0, CH)], iv2[p],
                                  smv[p]).wait()

        def fire_g(p):
            pltpu.async_copy(t1r.at[iv1[p]], gb1[p], smg[p])
            pltpu.async_copy(t2r.at[iv2[p]], gb2[p], smg[p])

        def wait_g(p):
            pltpu.make_async_copy(t1r.at[iv1[p]], gb1[p], smg[p]).wait()
            pltpu.make_async_copy(t2r.at[iv2[p]], gb2[p], smg[p]).wait()

        def fire_w(p, c):
            off = base0 + c * CH
            pltpu.async_copy(gb1[p], o1r.at[pl.ds(off, CH)], smw[p])
            pltpu.async_copy(gb2[p], o2r.at[pl.ds(off, CH)], smw[p])

        def wait_w(p):
            pltpu.make_async_copy(gb1[p], o1r.at[pl.ds(base0, CH)],
                                  smw[p]).wait()
            pltpu.make_async_copy(gb2[p], o2r.at[pl.ds(base0, CH)],
                                  smw[p]).wait()

        fire_i(0, 0)

        def body(i, carry):
            a = 2 * i
            wait_i(0)
            fire_i(1, a + 1)
            fire_g(0)
            wait_g(0)
            fire_w(0, a)
            wait_i(1)

            @pl.when(i > 0)
            def _():
                wait_w(1)

            fire_g(1)

            @pl.when(i + 1 < G2)
            def _():
                fire_i(0, a + 2)

            wait_g(1)
            fire_w(1, a + 1)
            wait_w(0)
            return carry

        lax.fori_loop(0, G2, body, 0)
        wait_w(1)
        if tail:
            c = nch - 1
            fire_i(0, c)
            wait_i(0)
            fire_g(0)
            wait_g(0)
            fire_w(0, c)
            wait_w(0)

    return k(t1, i1f, t2, i2f)


def _sc_scatter(vals, idx_flat, npad):
    E = idx_flat.shape[0]
    rows = E // NW
    nch = rows // CH
    assert vals.shape[0] == E and npad % 128 == 0, (vals.shape, E, npad)
    G2 = nch // 2
    tail = nch % 2 == 1
    zr = npad // 16
    zrows = jnp.zeros((zr, H), F32)
    ones = jnp.ones((CH, H), F32)

    scratch = [pltpu.VMEM((CH, H), F32) for _ in range(2)]
    scratch += [pltpu.VMEM((CH,), I32) for _ in range(2)]
    scratch += [pltpu.VMEM((CH, H), F32)]
    scratch += [pltpu.VMEM_SHARED((npad, H), F32)]
    scratch += [pltpu.SemaphoreType.DMA] * 4

    @functools.partial(
        pl.kernel, mesh=_sc_mesh(),
        out_type=(jax.ShapeDtypeStruct((2, npad, H), F32),
                  jax.ShapeDtypeStruct((2, npad, H), F32)),
        scratch_types=scratch,
    )
    def k(valsr, idxr, zrr, onesr, sumsr, cntr, *rest):
        vb = rest[0:2]
        ivc = rest[2:4]
        ob = rest[4]
        acc = rest[5]
        smv = rest[6:8]
        sma = rest[8:10]
        cid = lax.axis_index("c")
        sid = lax.axis_index("s")
        wid = sid * 2 + cid
        base0 = wid * rows

        def zero_acc():
            pltpu.sync_copy(zrr, acc.at[pl.ds(sid * zr, zr)])

        def fire_v(p, c, with_vals):
            off = base0 + c * CH
            if with_vals:
                pltpu.async_copy(valsr.at[pl.ds(off, CH)], vb[p], smv[p])
            pltpu.async_copy(idxr.at[pl.ds(off, CH)], ivc[p], smv[p])

        def wait_v(p, with_vals):
            if with_vals:
                pltpu.make_async_copy(valsr.at[pl.ds(base0, CH)], vb[p],
                                      smv[p]).wait()
            pltpu.make_async_copy(idxr.at[pl.ds(base0, CH)], ivc[p],
                                  smv[p]).wait()

        def fire_a(p, src):
            pltpu.async_copy(src, acc.at[ivc[p]], sma[p], add=True)

        def wait_a(p, src):
            pltpu.make_async_copy(src, acc.at[ivc[p]], sma[p]).wait()

        def phase(with_vals, outr):
            zero_acc()
            plsc.subcore_barrier()
            src = (lambda p: vb[p]) if with_vals else (lambda p: ob)
            fire_v(0, 0, with_vals)

            def body(i, carry):
                a = 2 * i
                wait_v(0, with_vals)
                fire_v(1, a + 1, with_vals)
                fire_a(0, src(0))
                wait_v(1, with_vals)
                wait_a(0, src(0))

                @pl.when(i + 1 < G2)
                def _():
                    fire_v(0, a + 2, with_vals)

                fire_a(1, src(1))
                wait_a(1, src(1))
                return carry

            lax.fori_loop(0, G2, body, 0)
            if tail:
                c = nch - 1
                fire_v(0, c, with_vals)
                wait_v(0, with_vals)
                fire_a(0, src(0))
                wait_a(0, src(0))
            plsc.subcore_barrier()
            pltpu.sync_copy(acc.at[pl.ds(sid * zr, zr)],
                            outr.at[cid, pl.ds(sid * zr, zr)])

        pltpu.sync_copy(onesr, ob)
        phase(True, sumsr)
        plsc.subcore_barrier()
        phase(False, cntr)

    return k(vals, idx_flat, zrows, ones)


def _unpack_counts(cntp, n):
    return cntp[0, :n, :1] + cntp[1, :n, :1]


def _pad_to(x, n, value):
    if x.shape[0] == n:
        return x
    pad = [(0, n - x.shape[0])] + [(0, 0)] * (x.ndim - 1)
    return jnp.pad(x, pad, constant_values=value)


def _idx3(col, ep, fill=None):
    e = col.shape[0]
    if e != ep:
        if fill is None:
            col = jnp.concatenate(
                [col, (jnp.arange(ep - e, dtype=col.dtype) % _NG)])
        else:
            col = _pad_to(col, ep, fill)
    return col.reshape(NW, ep // (NW * CH), CH)


def kernel(pc2g_edge_idx, pc2g_edge_features, pc2g_node_features,
           g2g_edge_idx, g2g_edge_features, g2pc_edge_idx, g2pc_edge_features,
           params):
    NPC = pc2g_node_features.shape[0]
    NG = _NG
    NGP = ((NG + 1 + 127) // 128) * 128
    NPCP = ((NPC + 1 + 127) // 128) * 128
    E1 = pc2g_edge_idx.shape[0]
    E2 = g2g_edge_idx.shape[0]
    E3 = g2pc_edge_idx.shape[0]
    E1P = ((E1 + _GRAN - 1) // _GRAN) * _GRAN
    E2P = ((E2 + _GRAN - 1) // _GRAN) * _GRAN
    E3P = ((E3 + _GRAN - 1) // _GRAN) * _GRAN
    gmod = globals()
    gather_pair = gmod["_sc_gather_pair"]
    scatter = gmod["_sc_scatter"]

    enc, proc, dec = params["enc"], params["proc"], params["dec"]

    ef1 = _pad_to(pc2g_edge_features, E1P, 0.0)
    n_full = _ffb(_pad_to(pc2g_node_features, NPCP, 0.0), enc["embed_node"])
    s1g = _idx3(pc2g_edge_idx[:, 0], E1P)
    r1g = _idx3(pc2g_edge_idx[:, 1], E1P)
    r1f = _pad_to(pc2g_edge_idx[:, 1], E1P, NG)
    ns, nr = gather_pair(n_full, s1g, n_full, r1g)
    e2 = _edge_mlp_embed(ns, nr, ef1, enc["embed_edge"], enc["gn_edge"])
    sums, cntp1 = scatter(e2, r1f, NGP)
    n_lat = _enc_node(n_full[:NGP], sums[0], sums[1],
                      _unpack_counts(cntp1, NGP),
                      enc["gn_node"], enc["out"])

    pe = _ffb(_pad_to(g2g_edge_features, E2P, 0.0), proc["embed_edge"])
    s2g = _idx3(g2g_edge_idx[:, 0], E2P)
    r2g = _idx3(g2g_edge_idx[:, 1], E2P)
    r2f = _pad_to(g2g_edge_idx[:, 1], E2P, NG)

    def round_fn(carry, gp):
        nl, pev = carry
        ns2, nr2 = gather_pair(nl, s2g, nl, r2g)
        ne, pev = _proc_edge_mlp(ns2, nr2, pev, gp["edge"])
        sums2, cntp2 = scatter(ne, r2f, NGP)
        nl = _proc_node(nl, sums2[0], sums2[1],
                        _unpack_counts(cntp2, NGP), gp["node"])
        return (nl, pev), None

    stacked = jax.tree.map(lambda *xs: jnp.stack(xs), *proc["gn"])
    (n_lat, pe), _ = lax.scan(round_fn, (n_lat, pe), stacked)
    n_lat = _ffb(n_lat, proc["out"])

    ef3 = _pad_to(g2pc_edge_features, E3P, 0.0)
    s3g = _idx3(g2pc_edge_idx[:, 0], E3P)
    r3g = _idx3(g2pc_edge_idx[:, 1], E3P)
    r3f = _pad_to(g2pc_edge_idx[:, 1], E3P, NPC)
    ns3, nr3 = gather_pair(n_lat, s3g, n_full, r3g)
    de2 = _edge_mlp_embed(ns3, nr3, ef3, dec["embed_edge"],
                          dec["proc_edge"])
    sums3, cntp3 = scatter(de2, r3f, NPCP)
    out = _dec_node(n_full[:NPC], sums3[0, :NPC], sums3[1, :NPC],
                    _unpack_counts(cntp3, NPC),
                    dec["proc_node"], dec["out"])
    return out
